# Initial kernel scaffold; baseline (speedup 1.0000x reference)
import jax
import jax.numpy as jnp
from jax.experimental import pallas as pl


def kernel(word, pos1, pos2, tag, MDPword, MDPpos, MDPrel, MDPdir, head, tail, root, word_table, pos1_table, pos2_table, tag_table, dir_table, deprel_table):
    raise NotImplementedError("write your pallas kernel here")


# speedup vs baseline: 3.5253x; 3.5253x over previous
import functools

import jax
import jax.numpy as jnp
from jax import lax
from jax.experimental import pallas as pl
from jax.experimental.pallas import tpu as pltpu
from jax.experimental.pallas import tpu_sc as plsc

B, L, LM = 1024, 200, 20
V, DW = 100000, 128
PS, TS, DS, RS = 32, 32, 16, 32

NC, NS = 2, 16
NW = NC * NS
SEN_ROWS = B * L
MDP_ROWS = B * LM
CH = 128
SEN_CHUNKS = SEN_ROWS // (NW * CH)
MDP_CHUNKS = MDP_ROWS // (NW * CH)
HT_PER_W = B // NW

SEN_D = DW + PS + PS + TS
MDP_D = DW + TS + RS + DS


def _body(word_i, p1_i, p2_i, tg_i, mw_i, mp_i, mr_i, md_i, hd_i, tl_i,
          wtab, p1tab, p2tab, tgtab, dtab, rtab,
          sen_o, mdp_o, h_o, t_o,
          widx, p1idx, p2idx, tgidx,
          mwidx, mpidx, mridx, mdidx,
          hidx, tidx,
          wbuf, p1buf, p2buf, tgbuf, mdbuf, htbuf,
          gsem):
    wid = lax.axis_index("s") * NC + lax.axis_index("c")

    pltpu.sync_copy(word_i.at[pl.ds(wid * SEN_CHUNKS, SEN_CHUNKS)], widx)
    pltpu.sync_copy(p1_i.at[pl.ds(wid * SEN_CHUNKS, SEN_CHUNKS)], p1idx)
    pltpu.sync_copy(p2_i.at[pl.ds(wid * SEN_CHUNKS, SEN_CHUNKS)], p2idx)
    pltpu.sync_copy(tg_i.at[pl.ds(wid * SEN_CHUNKS, SEN_CHUNKS)], tgidx)
    pltpu.sync_copy(mw_i.at[pl.ds(wid * MDP_CHUNKS, MDP_CHUNKS)], mwidx)
    pltpu.sync_copy(mp_i.at[pl.ds(wid * MDP_CHUNKS, MDP_CHUNKS)], mpidx)
    pltpu.sync_copy(mr_i.at[pl.ds(wid * MDP_CHUNKS, MDP_CHUNKS)], mridx)
    pltpu.sync_copy(md_i.at[pl.ds(wid * MDP_CHUNKS, MDP_CHUNKS)], mdidx)
    pltpu.sync_copy(hd_i.at[wid], hidx)
    pltpu.sync_copy(tl_i.at[wid], tidx)

    sen_base = wid * SEN_CHUNKS * CH

    def sen_step(c, _):
        r0 = sen_base + c * CH
        cw = pltpu.async_copy(wtab.at[widx.at[c]], wbuf, gsem)
        c1 = pltpu.async_copy(p1tab.at[p1idx.at[c]], p1buf, gsem)
        c2 = pltpu.async_copy(p2tab.at[p2idx.at[c]], p2buf, gsem)
        c3 = pltpu.async_copy(tgtab.at[tgidx.at[c]], tgbuf, gsem)
        cw.wait(); c1.wait(); c2.wait(); c3.wait()
        pltpu.sync_copy(wbuf, sen_o.at[pl.ds(r0, CH), pl.ds(0, DW)])
        pltpu.sync_copy(p1buf, sen_o.at[pl.ds(r0, CH), pl.ds(DW, PS)])
        pltpu.sync_copy(p2buf, sen_o.at[pl.ds(r0, CH), pl.ds(DW + PS, PS)])
        pltpu.sync_copy(tgbuf, sen_o.at[pl.ds(r0, CH), pl.ds(DW + 2 * PS, TS)])
        return _

    lax.fori_loop(0, SEN_CHUNKS, sen_step, None)

    mdp_base = wid * MDP_CHUNKS * CH

    def mdp_step(c, _):
        r0 = mdp_base + c * CH
        cw = pltpu.async_copy(wtab.at[mwidx.at[c]], wbuf, gsem)
        c1 = pltpu.async_copy(tgtab.at[mpidx.at[c]], p1buf, gsem)
        c2 = pltpu.async_copy(rtab.at[mridx.at[c]], p2buf, gsem)
        c3 = pltpu.async_copy(dtab.at[mdidx.at[c]], mdbuf, gsem)
        cw.wait(); c1.wait(); c2.wait(); c3.wait()
        pltpu.sync_copy(wbuf, mdp_o.at[pl.ds(r0, CH), pl.ds(0, DW)])
        pltpu.sync_copy(p1buf, mdp_o.at[pl.ds(r0, CH), pl.ds(DW, TS)])
        pltpu.sync_copy(p2buf, mdp_o.at[pl.ds(r0, CH), pl.ds(DW + TS, RS)])
        pltpu.sync_copy(mdbuf, mdp_o.at[pl.ds(r0, CH), pl.ds(DW + TS + RS, DS)])
        return _

    lax.fori_loop(0, MDP_CHUNKS, mdp_step, None)

    hb = wid * HT_PER_W
    pltpu.async_copy(wtab.at[hidx], htbuf, gsem).wait()
    pltpu.sync_copy(htbuf, h_o.at[pl.ds(hb, HT_PER_W)])
    pltpu.async_copy(wtab.at[tidx], htbuf, gsem).wait()
    pltpu.sync_copy(htbuf, t_o.at[pl.ds(hb, HT_PER_W)])


@jax.jit
def _run(word, pos1, pos2, tag, mdpw, mdpp, mdpr, mdpd, head, tail,
         word_table, pos1_table, pos2_table, tag_table, dir_table, deprel_table):
    i32 = jnp.int32
    word2d = word.reshape(SEN_ROWS // CH, CH).astype(i32)
    p12d = pos1.reshape(SEN_ROWS // CH, CH).astype(i32)
    p22d = pos2.reshape(SEN_ROWS // CH, CH).astype(i32)
    tg2d = tag.reshape(SEN_ROWS // CH, CH).astype(i32)
    mw2d = mdpw.reshape(MDP_ROWS // CH, CH).astype(i32)
    mp2d = mdpp.reshape(MDP_ROWS // CH, CH).astype(i32)
    mr2d = mdpr.reshape(MDP_ROWS // CH, CH).astype(i32)
    md2d = mdpd.reshape(MDP_ROWS // CH, CH).astype(i32)
    hd2d = head.reshape(NW, HT_PER_W).astype(i32)
    tl2d = tail.reshape(NW, HT_PER_W).astype(i32)

    mesh = plsc.VectorSubcoreMesh(core_axis_name="c", subcore_axis_name="s",
                                  num_cores=NC, num_subcores=NS)
    f32 = jnp.float32
    k = pl.kernel(
        _body,
        out_type=(
            jax.ShapeDtypeStruct((SEN_ROWS, SEN_D), f32),
            jax.ShapeDtypeStruct((MDP_ROWS, MDP_D), f32),
            jax.ShapeDtypeStruct((B, DW), f32),
            jax.ShapeDtypeStruct((B, DW), f32),
        ),
        mesh=mesh,
        scratch_types=(
            pltpu.VMEM((SEN_CHUNKS, CH), i32),
            pltpu.VMEM((SEN_CHUNKS, CH), i32),
            pltpu.VMEM((SEN_CHUNKS, CH), i32),
            pltpu.VMEM((SEN_CHUNKS, CH), i32),
            pltpu.VMEM((MDP_CHUNKS, CH), i32),
            pltpu.VMEM((MDP_CHUNKS, CH), i32),
            pltpu.VMEM((MDP_CHUNKS, CH), i32),
            pltpu.VMEM((MDP_CHUNKS, CH), i32),
            pltpu.VMEM((HT_PER_W,), i32),
            pltpu.VMEM((HT_PER_W,), i32),
            pltpu.VMEM((CH, DW), f32),
            pltpu.VMEM((CH, PS), f32),
            pltpu.VMEM((CH, PS), f32),
            pltpu.VMEM((CH, TS), f32),
            pltpu.VMEM((CH, DS), f32),
            pltpu.VMEM((HT_PER_W, DW), f32),
            pltpu.SemaphoreType.DMA,
        ),
        compiler_params=pltpu.CompilerParams(use_tc_tiling_on_sc=False),
    )
    sen2d, mdp2d, h, t = k(word2d, p12d, p22d, tg2d, mw2d, mp2d, mr2d, md2d,
                           hd2d, tl2d, word_table, pos1_table, pos2_table,
                           tag_table, dir_table, deprel_table)
    return sen2d.reshape(B, L, SEN_D), mdp2d.reshape(B, LM, MDP_D), h, t


def kernel(word, pos1, pos2, tag, MDPword, MDPpos, MDPrel, MDPdir, head, tail,
           root, word_table, pos1_table, pos2_table, tag_table, dir_table,
           deprel_table):
    return _run(word, pos1, pos2, tag, MDPword, MDPpos, MDPrel, MDPdir,
                head, tail, word_table, pos1_table, pos2_table, tag_table,
                dir_table, deprel_table)


# speedup vs baseline: 3.5798x; 1.0155x over previous
import jax
import jax.numpy as jnp
from jax import lax
from jax.experimental import pallas as pl
from jax.experimental.pallas import tpu as pltpu
from jax.experimental.pallas import tpu_sc as plsc

B, L, LM = 1024, 200, 20
V, DW = 100000, 128
PS, TS, DS, RS = 32, 32, 16, 32

NC, NS = 2, 16
NW = NC * NS
SEN_ROWS = B * L
MDP_ROWS = B * LM
CH = 128
SEN_CHUNKS = SEN_ROWS // (NW * CH)
MDP_CHUNKS = MDP_ROWS // (NW * CH)
HT_PER_W = B // NW

SEN_D = DW + PS + PS + TS
MDP_D = DW + TS + RS + DS


def _body(word_i, p1_i, p2_i, tg_i, mw_i, mp_i, mr_i, md_i, hd_i, tl_i,
          wtab, p1tab, p2tab, tgtab, dtab, rtab,
          sen_o, mdp_o, h_o, t_o,
          widx, p1idx, p2idx, tgidx,
          mwidx, mpidx, mridx, mdidx,
          hidx, tidx,
          wbufA, p1bufA, p2bufA, tgbufA,
          wbufB, p1bufB, p2bufB, tgbufB,
          mdbuf, htbuf,
          gsemA, gsemB, osemA, osemB):
    wid = lax.axis_index("s") * NC + lax.axis_index("c")

    pltpu.sync_copy(word_i.at[pl.ds(wid * SEN_CHUNKS, SEN_CHUNKS)], widx)
    pltpu.sync_copy(p1_i.at[pl.ds(wid * SEN_CHUNKS, SEN_CHUNKS)], p1idx)
    pltpu.sync_copy(p2_i.at[pl.ds(wid * SEN_CHUNKS, SEN_CHUNKS)], p2idx)
    pltpu.sync_copy(tg_i.at[pl.ds(wid * SEN_CHUNKS, SEN_CHUNKS)], tgidx)
    pltpu.sync_copy(mw_i.at[pl.ds(wid * MDP_CHUNKS, MDP_CHUNKS)], mwidx)
    pltpu.sync_copy(mp_i.at[pl.ds(wid * MDP_CHUNKS, MDP_CHUNKS)], mpidx)
    pltpu.sync_copy(mr_i.at[pl.ds(wid * MDP_CHUNKS, MDP_CHUNKS)], mridx)
    pltpu.sync_copy(md_i.at[pl.ds(wid * MDP_CHUNKS, MDP_CHUNKS)], mdidx)
    pltpu.sync_copy(hd_i.at[wid], hidx)
    pltpu.sync_copy(tl_i.at[wid], tidx)

    bufA = (wbufA, p1bufA, p2bufA, tgbufA)
    bufB = (wbufB, p1bufB, p2bufB, tgbufB)
    sen_base = wid * SEN_CHUNKS * CH

    def sen_gather_descs(c, bufs, sem):
        return (pltpu.make_async_copy(wtab.at[widx.at[c]], bufs[0], sem),
                pltpu.make_async_copy(p1tab.at[p1idx.at[c]], bufs[1], sem),
                pltpu.make_async_copy(p2tab.at[p2idx.at[c]], bufs[2], sem),
                pltpu.make_async_copy(tgtab.at[tgidx.at[c]], bufs[3], sem))

    def sen_write_descs(c, bufs, sem):
        r0 = sen_base + c * CH
        return (
            pltpu.make_async_copy(bufs[0], sen_o.at[pl.ds(r0, CH), pl.ds(0, DW)], sem),
            pltpu.make_async_copy(bufs[1], sen_o.at[pl.ds(r0, CH), pl.ds(DW, PS)], sem),
            pltpu.make_async_copy(bufs[2], sen_o.at[pl.ds(r0, CH), pl.ds(DW + PS, PS)], sem),
            pltpu.make_async_copy(bufs[3], sen_o.at[pl.ds(r0, CH), pl.ds(DW + 2 * PS, TS)], sem),
        )

    def start(descs):
        for d in descs:
            d.start()

    def wait(descs):
        for d in descs:
            d.wait()

    HALF = SEN_CHUNKS // 2

    start(sen_gather_descs(0, bufA, gsemA))

    def sen_step(i, _):
        cA = 2 * i
        cB = cA + 1
        wait(sen_gather_descs(cA, bufA, gsemA))
        start(sen_write_descs(cA, bufA, osemA))

        @pl.when(i > 0)
        def _drainB():
            wait(sen_write_descs(cB, bufB, osemB))

        start(sen_gather_descs(cB, bufB, gsemB))
        wait(sen_gather_descs(cB, bufB, gsemB))
        start(sen_write_descs(cB, bufB, osemB))

        @pl.when(i < HALF - 1)
        def _nextA():
            wait(sen_write_descs(cA, bufA, osemA))
            start(sen_gather_descs(cA + 2, bufA, gsemA))

        return _

    lax.fori_loop(0, HALF, sen_step, None)
    wait(sen_write_descs(0, bufA, osemA))
    wait(sen_write_descs(0, bufB, osemB))

    mdp_base = wid * MDP_CHUNKS * CH
    mbufs = (wbufA, p1bufA, p2bufA, mdbuf)

    def mdp_gather_descs(c, sem):
        return (pltpu.make_async_copy(wtab.at[mwidx.at[c]], mbufs[0], sem),
                pltpu.make_async_copy(tgtab.at[mpidx.at[c]], mbufs[1], sem),
                pltpu.make_async_copy(rtab.at[mridx.at[c]], mbufs[2], sem),
                pltpu.make_async_copy(dtab.at[mdidx.at[c]], mbufs[3], sem))

    def mdp_write_descs(c, sem):
        r0 = mdp_base + c * CH
        return (
            pltpu.make_async_copy(mbufs[0], mdp_o.at[pl.ds(r0, CH), pl.ds(0, DW)], sem),
            pltpu.make_async_copy(mbufs[1], mdp_o.at[pl.ds(r0, CH), pl.ds(DW, TS)], sem),
            pltpu.make_async_copy(mbufs[2], mdp_o.at[pl.ds(r0, CH), pl.ds(DW + TS, RS)], sem),
            pltpu.make_async_copy(mbufs[3], mdp_o.at[pl.ds(r0, CH), pl.ds(DW + TS + RS, DS)], sem),
        )

    def mdp_step(c, _):
        @pl.when(c > 0)
        def _drain():
            wait(mdp_write_descs(c, osemA))

        start(mdp_gather_descs(c, gsemA))
        wait(mdp_gather_descs(c, gsemA))
        start(mdp_write_descs(c, osemA))
        return _

    lax.fori_loop(0, MDP_CHUNKS, mdp_step, None)
    wait(mdp_write_descs(0, osemA))

    hb = wid * HT_PER_W
    pltpu.async_copy(wtab.at[hidx], htbuf, gsemA).wait()
    pltpu.sync_copy(htbuf, h_o.at[pl.ds(hb, HT_PER_W)])
    pltpu.async_copy(wtab.at[tidx], htbuf, gsemA).wait()
    pltpu.sync_copy(htbuf, t_o.at[pl.ds(hb, HT_PER_W)])


@jax.jit
def _run(word, pos1, pos2, tag, mdpw, mdpp, mdpr, mdpd, head, tail,
         word_table, pos1_table, pos2_table, tag_table, dir_table, deprel_table):
    i32 = jnp.int32
    word2d = word.reshape(SEN_ROWS // CH, CH).astype(i32)
    p12d = pos1.reshape(SEN_ROWS // CH, CH).astype(i32)
    p22d = pos2.reshape(SEN_ROWS // CH, CH).astype(i32)
    tg2d = tag.reshape(SEN_ROWS // CH, CH).astype(i32)
    mw2d = mdpw.reshape(MDP_ROWS // CH, CH).astype(i32)
    mp2d = mdpp.reshape(MDP_ROWS // CH, CH).astype(i32)
    mr2d = mdpr.reshape(MDP_ROWS // CH, CH).astype(i32)
    md2d = mdpd.reshape(MDP_ROWS // CH, CH).astype(i32)
    hd2d = head.reshape(NW, HT_PER_W).astype(i32)
    tl2d = tail.reshape(NW, HT_PER_W).astype(i32)

    mesh = plsc.VectorSubcoreMesh(core_axis_name="c", subcore_axis_name="s",
                                  num_cores=NC, num_subcores=NS)
    f32 = jnp.float32
    k = pl.kernel(
        _body,
        out_type=(
            jax.ShapeDtypeStruct((SEN_ROWS, SEN_D), f32),
            jax.ShapeDtypeStruct((MDP_ROWS, MDP_D), f32),
            jax.ShapeDtypeStruct((B, DW), f32),
            jax.ShapeDtypeStruct((B, DW), f32),
        ),
        mesh=mesh,
        scratch_types=(
            pltpu.VMEM((SEN_CHUNKS, CH), i32),
            pltpu.VMEM((SEN_CHUNKS, CH), i32),
            pltpu.VMEM((SEN_CHUNKS, CH), i32),
            pltpu.VMEM((SEN_CHUNKS, CH), i32),
            pltpu.VMEM((MDP_CHUNKS, CH), i32),
            pltpu.VMEM((MDP_CHUNKS, CH), i32),
            pltpu.VMEM((MDP_CHUNKS, CH), i32),
            pltpu.VMEM((MDP_CHUNKS, CH), i32),
            pltpu.VMEM((HT_PER_W,), i32),
            pltpu.VMEM((HT_PER_W,), i32),
            pltpu.VMEM((CH, DW), f32),
            pltpu.VMEM((CH, PS), f32),
            pltpu.VMEM((CH, PS), f32),
            pltpu.VMEM((CH, TS), f32),
            pltpu.VMEM((CH, DW), f32),
            pltpu.VMEM((CH, PS), f32),
            pltpu.VMEM((CH, PS), f32),
            pltpu.VMEM((CH, TS), f32),
            pltpu.VMEM((CH, DS), f32),
            pltpu.VMEM((HT_PER_W, DW), f32),
            pltpu.SemaphoreType.DMA,
            pltpu.SemaphoreType.DMA,
            pltpu.SemaphoreType.DMA,
            pltpu.SemaphoreType.DMA,
        ),
        compiler_params=pltpu.CompilerParams(use_tc_tiling_on_sc=False),
    )
    sen2d, mdp2d, h, t = k(word2d, p12d, p22d, tg2d, mw2d, mp2d, mr2d, md2d,
                           hd2d, tl2d, word_table, pos1_table, pos2_table,
                           tag_table, dir_table, deprel_table)
    return sen2d.reshape(B, L, SEN_D), mdp2d.reshape(B, LM, MDP_D), h, t


def kernel(word, pos1, pos2, tag, MDPword, MDPpos, MDPrel, MDPdir, head, tail,
           root, word_table, pos1_table, pos2_table, tag_table, dir_table,
           deprel_table):
    return _run(word, pos1, pos2, tag, MDPword, MDPpos, MDPrel, MDPdir,
                head, tail, word_table, pos1_table, pos2_table, tag_table,
                dir_table, deprel_table)
